# baseline (speedup 1.0000x reference)
import functools

import jax
import jax.numpy as jnp
from jax import lax
from jax.experimental import pallas as pl
from jax.experimental.pallas import tpu as pltpu
from jax.experimental.pallas import tpu_sc as plsc

N_NODES = 10000
N_EDGES = 320000
D = 128

NC = 2
NS = 16
NW = NC * NS
EPW = N_EDGES // NW
CH = 104
NFULL = 96
REM = EPW - NFULL * CH
RPT = 624
RTAIL = N_NODES - NS * RPT
BT = 1000
LANES = 16
NRB = 3
NIB = 6
assert NFULL % NIB == 0 and NIB % NRB == 0

_SC_PARAMS = pltpu.CompilerParams(needs_layout_passes=False)


DZ = 640
DN = NS * DZ


@functools.cache
def _mk_sc_agg(with_deg: bool):
    mesh = plsc.VectorSubcoreMesh(core_axis_name="c", subcore_axis_name="s")
    out_type = [jax.ShapeDtypeStruct((NC, N_NODES, D), jnp.float32)]
    scratch = [
        pltpu.VMEM((NRB, CH, D), jnp.float32),
        pltpu.VMEM((NIB, CH), jnp.int32),
        pltpu.VMEM((NIB, CH), jnp.int32),
        pltpu.VMEM((1, REM), jnp.int32),
        pltpu.VMEM((1, REM), jnp.int32),
        pltpu.VMEM_SHARED((N_NODES, D), jnp.float32),
    ]
    nsem = 2 * NRB + 2 * NIB
    if with_deg:
        out_type.append(jax.ShapeDtypeStruct((NC, 1, DN), jnp.float32))
        scratch += [
            pltpu.VMEM((112,), jnp.float32),
            pltpu.VMEM((DZ,), jnp.float32),
            pltpu.VMEM_SHARED((DN,), jnp.float32),
        ]
        nsem += NRB
    scratch += [pltpu.SemaphoreType.DMA] * nsem

    def body(p_hbm, src_hbm, dstm_hbm, dstr_hbm, *rest):
        if with_deg:
            (s_out, deg_out, rows_v, srci_v, dsti_v, dstr_v, srcr_v, acc_sh,
             ones_v, zdeg_v, deg_sh, *sems) = rest
        else:
            (s_out, rows_v, srci_v, dsti_v, dstr_v, srcr_v, acc_sh,
             *sems) = rest
            deg_out = ones_v = zdeg_v = deg_sh = None
        gsem = sems[:NRB]
        ssem = sems[NRB:2 * NRB]
        isem = sems[2 * NRB:2 * NRB + NIB]
        dsem = sems[2 * NRB + NIB:2 * NRB + 2 * NIB]
        osem = sems[2 * NRB + 2 * NIB:]

        c = lax.axis_index("c")
        s = lax.axis_index("s")
        wid = s * NC + c
        base = wid * EPW

        pltpu.sync_copy(dstr_hbm.at[wid], dstr_v)
        pltpu.sync_copy(src_hbm.at[pl.ds(base + NFULL * CH, REM)], srcr_v.at[0])

        zeros16 = jnp.zeros((LANES,), jnp.float32)

        def zrow(i, _):
            for l in range(D // LANES):
                rows_v[0, i, pl.ds(l * LANES, LANES)] = zeros16
            return 0
        lax.fori_loop(0, CH, zrow, 0)

        assert RPT % CH == 0
        row0 = s * RPT
        for k in range(RPT // CH):
            pltpu.sync_copy(rows_v.at[0], acc_sh.at[pl.ds(row0 + k * CH, CH)])

        @pl.when(s == NS - 1)
        def _():
            pltpu.sync_copy(rows_v.at[0, pl.ds(0, RTAIL)],
                            acc_sh.at[pl.ds(NS * RPT, RTAIL)])

        if with_deg:
            ones16 = jnp.ones((LANES,), jnp.float32)
            for k in range(112 // LANES):
                ones_v[pl.ds(k * LANES, LANES)] = ones16
            for k in range(DZ // LANES):
                zdeg_v[pl.ds(k * LANES, LANES)] = zeros16

            pltpu.sync_copy(zdeg_v, deg_sh.at[pl.ds(s * DZ, DZ)])
        plsc.subcore_barrier()

        for u in range(NIB - 1):
            pltpu.async_copy(src_hbm.at[pl.ds(base + u * CH, CH)],
                             srci_v.at[u], isem[u])
        for u in range(NIB):
            pltpu.async_copy(dstm_hbm.at[wid, u], dsti_v.at[u], dsem[u])

        def _wait_scat(b):
            pltpu.make_async_copy(rows_v.at[b], acc_sh.at[dsti_v.at[0]],
                                  ssem[b]).wait()
            if with_deg:
                pltpu.make_async_copy(ones_v.at[pl.ds(0, CH)],
                                      deg_sh.at[dsti_v.at[0]],
                                      osem[b]).wait()

        def _wait_gath(b):
            pltpu.make_async_copy(p_hbm.at[srci_v.at[0]], rows_v.at[b],
                                  gsem[b]).wait()

        def _wait_idx(u):
            pltpu.make_async_copy(src_hbm.at[pl.ds(base, CH)], srci_v.at[u],
                                  isem[u]).wait()

        def _wait_didx(u):
            pltpu.make_async_copy(dstm_hbm.at[wid, 0], dsti_v.at[u],
                                  dsem[u]).wait()

        def step(t, u):
            j = t * NIB + u
            rb = u % NRB
            fd = (u + NRB) % NIB

            def scat_done():
                _wait_scat(rb)

                @pl.when(j + NRB < NFULL)
                def _():
                    pltpu.async_copy(dstm_hbm.at[wid, j + NRB],
                                     dsti_v.at[fd], dsem[fd])
            if u >= NRB:
                scat_done()
            else:
                @pl.when(t > 0)
                def _():
                    scat_done()
            _wait_idx(u)
            pltpu.async_copy(p_hbm.at[srci_v.at[u]], rows_v.at[rb], gsem[rb])
            pb = (u + NRB - 1) % NRB
            pu = (u + NIB - 1) % NIB

            def fire_scat():
                _wait_gath(pb)
                _wait_didx(pu)
                pltpu.async_copy(rows_v.at[pb], acc_sh.at[dsti_v.at[pu]],
                                 ssem[pb], add=True)
                if with_deg:
                    pltpu.async_copy(ones_v.at[pl.ds(0, CH)],
                                     deg_sh.at[dsti_v.at[pu]],
                                     osem[pb], add=True)
            if u >= 1:
                fire_scat()
            else:
                @pl.when(t > 0)
                def _():
                    fire_scat()
            @pl.when(j + NIB - 1 < NFULL)
            def _():
                off = pl.multiple_of((j + NIB - 1) * CH, 8)
                pltpu.async_copy(src_hbm.at[pl.ds(base + off, CH)],
                                 srci_v.at[pu], isem[pu])

        def group(t, _):
            for u in range(NIB):
                step(t, u)
            return 0
        lax.fori_loop(0, NFULL // NIB, group, 0)

        lastb = (NFULL - 1) % NRB
        lastu = (NFULL - 1) % NIB
        _wait_gath(lastb)
        _wait_didx(lastu)
        pltpu.async_copy(rows_v.at[lastb], acc_sh.at[dsti_v.at[lastu]],
                         ssem[lastb], add=True)
        if with_deg:
            pltpu.async_copy(ones_v.at[pl.ds(0, CH)],
                             deg_sh.at[dsti_v.at[lastu]], osem[lastb],
                             add=True)
        for b in range(NRB):
            _wait_scat(b)

        pltpu.async_copy(p_hbm.at[srcr_v.at[0]],
                         rows_v.at[0, pl.ds(0, REM)], gsem[0])
        pltpu.make_async_copy(p_hbm.at[srcr_v.at[0]],
                              rows_v.at[0, pl.ds(0, REM)], gsem[0]).wait()
        pltpu.sync_copy(rows_v.at[0, pl.ds(0, REM)],
                        acc_sh.at[dstr_v.at[0]], add=True)
        if with_deg:
            pltpu.sync_copy(ones_v.at[pl.ds(0, REM)],
                            deg_sh.at[dstr_v.at[0]], add=True)

        plsc.subcore_barrier()
        pltpu.sync_copy(acc_sh.at[pl.ds(row0, RPT)],
                        s_out.at[c, pl.ds(row0, RPT)])

        @pl.when(s == NS - 1)
        def _():
            pltpu.sync_copy(acc_sh.at[pl.ds(NS * RPT, RTAIL)],
                            s_out.at[c, pl.ds(NS * RPT, RTAIL)])

        if with_deg:
            pltpu.sync_copy(deg_sh.at[pl.ds(s * DZ, DZ)],
                            deg_out.at[c, 0, pl.ds(s * DZ, DZ)])

    return pl.kernel(body, mesh=mesh, out_type=out_type,
                     scratch_types=scratch, compiler_params=_SC_PARAMS)


_DOT = functools.partial(jnp.dot, preferred_element_type=jnp.float32,
                         precision=lax.Precision.HIGHEST)


def _tc1_body(x_ref, wt_ref, wb_ref, b_ref, a_ref, p_ref):
    xb = x_ref[...]
    a_ref[...] = _DOT(xb, wt_ref[...]) + b_ref[...]
    p_ref[...] = _DOT(xb, wb_ref[...])


def _tc2_body(a1_ref, s_ref, deg_ref, wt_ref, wb_ref, b_ref, a2_ref, p2_ref):
    ssum = s_ref[0] + s_ref[1]
    deg = jnp.maximum(jnp.sum(deg_ref[...], axis=1, keepdims=True), 1.0)
    h = jnp.maximum(a1_ref[...] + ssum / deg, 0.0)
    a2_ref[...] = _DOT(h, wt_ref[...]) + b_ref[...]
    p2_ref[...] = _DOT(h, wb_ref[...])


def _tc3_body(a2_ref, s_ref, deg_ref, o_ref):
    ssum = s_ref[0] + s_ref[1]
    deg = jnp.maximum(jnp.sum(deg_ref[...], axis=1, keepdims=True), 1.0)
    o_ref[...] = jnp.maximum(a2_ref[...] + ssum / deg, 0.0)


_ROWS = pl.BlockSpec((BT, D), lambda i: (i, 0))
_WMAT = pl.BlockSpec((D, D), lambda i: (0, 0))
_BIAS = pl.BlockSpec((1, D), lambda i: (0, 0))
_SPART = pl.BlockSpec((NC, BT, D), lambda i: (0, i, 0))
_DEGP = pl.BlockSpec((BT, NC), lambda i: (i, 0))
_GRID = (N_NODES // BT,)
_ND = jax.ShapeDtypeStruct((N_NODES, D), jnp.float32)


def _tc1(x, wt, wb, b):
    return pl.pallas_call(
        _tc1_body, grid=_GRID,
        in_specs=[_ROWS, _WMAT, _WMAT, _BIAS],
        out_specs=[_ROWS, _ROWS], out_shape=[_ND, _ND],
    )(x, wt, wb, b)


def _tc2(a1, s_parts, deg_parts, wt, wb, b):
    return pl.pallas_call(
        _tc2_body, grid=_GRID,
        in_specs=[_ROWS, _SPART, _DEGP, _WMAT, _WMAT, _BIAS],
        out_specs=[_ROWS, _ROWS], out_shape=[_ND, _ND],
    )(a1, s_parts, deg_parts, wt, wb, b)


def _tc3(a2, s_parts, deg_parts):
    return pl.pallas_call(
        _tc3_body, grid=_GRID,
        in_specs=[_ROWS, _SPART, _DEGP],
        out_specs=_ROWS, out_shape=_ND,
    )(a2, s_parts, deg_parts)


def kernel(x, edge_index, W1, b1, W2, b2):
    src = edge_index[0].astype(jnp.int32)
    dst = edge_index[1].astype(jnp.int32)
    dst2 = dst.reshape(NW, EPW)
    dst_main = dst2[:, :NFULL * CH].reshape(NW, NFULL, CH)
    dst_rem = dst2[:, NFULL * CH:].reshape(NW, 1, REM)
    b1r = b1.reshape(1, D)
    b2r = b2.reshape(1, D)

    a1, p1 = _tc1(x, W1[:D], W1[D:], b1r)
    s1, degp = _mk_sc_agg(True)(p1, src, dst_main, dst_rem)
    degp = degp.reshape(NC, DN)[:, :N_NODES].T
    a2, p2 = _tc2(a1, s1, degp, W2[:D], W2[D:], b2r)
    (s2,) = _mk_sc_agg(False)(p2, src, dst_main, dst_rem)
    return _tc3(a2, s2, degp)


# speedup vs baseline: 1.0953x; 1.0953x over previous
import functools

import jax
import jax.numpy as jnp
from jax import lax
from jax.experimental import pallas as pl
from jax.experimental.pallas import tpu as pltpu
from jax.experimental.pallas import tpu_sc as plsc

N_NODES = 10000
N_EDGES = 320000
D = 128

NC = 2
NS = 16
NW = NC * NS
EPW = N_EDGES // NW
CH = 104
NFULL = 96
REM = EPW - NFULL * CH
RPT = 624
RTAIL = N_NODES - NS * RPT
BT = 1000
LANES = 16
NRB = 3
NIB = 6
assert NFULL % NIB == 0 and NIB % NRB == 0

_SC_PARAMS = pltpu.CompilerParams(needs_layout_passes=False)


DZ = 640
DN = NS * DZ


@functools.cache
def _mk_sc_agg(with_deg: bool):
    mesh = plsc.VectorSubcoreMesh(core_axis_name="c", subcore_axis_name="s")
    out_type = [jax.ShapeDtypeStruct((NC, N_NODES, D), jnp.float32)]
    scratch = [
        pltpu.VMEM((NRB, CH, D), jnp.float32),
        pltpu.VMEM((NIB, CH), jnp.int32),
        pltpu.VMEM((NIB, CH), jnp.int32),
        pltpu.VMEM((1, REM), jnp.int32),
        pltpu.VMEM((1, REM), jnp.int32),
        pltpu.VMEM_SHARED((N_NODES, D), jnp.float32),
    ]
    nsem = 2 * NRB + 2 * NIB
    if with_deg:
        out_type.append(jax.ShapeDtypeStruct((NC, 1, DN), jnp.float32))
        scratch += [
            pltpu.VMEM((112,), jnp.float32),
            pltpu.VMEM((DZ,), jnp.float32),
            pltpu.VMEM_SHARED((DN,), jnp.float32),
        ]
        nsem += NRB
    scratch += [pltpu.SemaphoreType.DMA] * nsem

    def body(p_hbm, src_hbm, dst_hbm, *rest):
        if with_deg:
            (s_out, deg_out, rows_v, srci_v, dsti_v, dstr_v, srcr_v, acc_sh,
             ones_v, zdeg_v, deg_sh, *sems) = rest
        else:
            (s_out, rows_v, srci_v, dsti_v, dstr_v, srcr_v, acc_sh,
             *sems) = rest
            deg_out = ones_v = zdeg_v = deg_sh = None
        gsem = sems[:NRB]
        ssem = sems[NRB:2 * NRB]
        isem = sems[2 * NRB:2 * NRB + NIB]
        dsem = sems[2 * NRB + NIB:2 * NRB + 2 * NIB]
        osem = sems[2 * NRB + 2 * NIB:]

        c = lax.axis_index("c")
        s = lax.axis_index("s")
        wid = s * NC + c
        base = wid * EPW

        pltpu.sync_copy(dst_hbm.at[pl.ds(base + NFULL * CH, REM)], dstr_v.at[0])
        pltpu.sync_copy(src_hbm.at[pl.ds(base + NFULL * CH, REM)], srcr_v.at[0])

        zeros16 = jnp.zeros((LANES,), jnp.float32)

        def zrow(i, _):
            for l in range(D // LANES):
                rows_v[0, i, pl.ds(l * LANES, LANES)] = zeros16
            return 0
        lax.fori_loop(0, CH, zrow, 0)

        assert RPT % CH == 0
        row0 = s * RPT
        for k in range(RPT // CH):
            pltpu.sync_copy(rows_v.at[0], acc_sh.at[pl.ds(row0 + k * CH, CH)])

        @pl.when(s == NS - 1)
        def _():
            pltpu.sync_copy(rows_v.at[0, pl.ds(0, RTAIL)],
                            acc_sh.at[pl.ds(NS * RPT, RTAIL)])

        if with_deg:
            ones16 = jnp.ones((LANES,), jnp.float32)
            for k in range(112 // LANES):
                ones_v[pl.ds(k * LANES, LANES)] = ones16
            for k in range(DZ // LANES):
                zdeg_v[pl.ds(k * LANES, LANES)] = zeros16

            pltpu.sync_copy(zdeg_v, deg_sh.at[pl.ds(s * DZ, DZ)])
        plsc.subcore_barrier()

        for u in range(NIB - 1):
            pltpu.async_copy(src_hbm.at[pl.ds(base + u * CH, CH)],
                             srci_v.at[u], isem[u])
        for u in range(NIB):
            pltpu.async_copy(dst_hbm.at[pl.ds(base + u * CH, CH)],
                             dsti_v.at[u], dsem[u])

        def _wait_scat(b):
            pltpu.make_async_copy(rows_v.at[b], acc_sh.at[dsti_v.at[0]],
                                  ssem[b]).wait()
            if with_deg:
                pltpu.make_async_copy(ones_v.at[pl.ds(0, CH)],
                                      deg_sh.at[dsti_v.at[0]],
                                      osem[b]).wait()

        def _wait_gath(b):
            pltpu.make_async_copy(p_hbm.at[srci_v.at[0]], rows_v.at[b],
                                  gsem[b]).wait()

        def _wait_idx(u):
            pltpu.make_async_copy(src_hbm.at[pl.ds(base, CH)], srci_v.at[u],
                                  isem[u]).wait()

        def _wait_didx(u):
            pltpu.make_async_copy(dst_hbm.at[pl.ds(base, CH)], dsti_v.at[u],
                                  dsem[u]).wait()

        def step(t, u):
            j = t * NIB + u
            rb = u % NRB
            fd = (u + NRB) % NIB

            def scat_done():
                _wait_scat(rb)

                @pl.when(j + NRB < NFULL)
                def _():
                    doff = pl.multiple_of((j + NRB) * CH, 8)
                    pltpu.async_copy(dst_hbm.at[pl.ds(base + doff, CH)],
                                     dsti_v.at[fd], dsem[fd])
            if u >= NRB:
                scat_done()
            else:
                @pl.when(t > 0)
                def _():
                    scat_done()
            _wait_idx(u)
            pltpu.async_copy(p_hbm.at[srci_v.at[u]], rows_v.at[rb], gsem[rb])
            pb = (u + NRB - 1) % NRB
            pu = (u + NIB - 1) % NIB

            def fire_scat():
                _wait_gath(pb)
                _wait_didx(pu)
                pltpu.async_copy(rows_v.at[pb], acc_sh.at[dsti_v.at[pu]],
                                 ssem[pb], add=True)
                if with_deg:
                    pltpu.async_copy(ones_v.at[pl.ds(0, CH)],
                                     deg_sh.at[dsti_v.at[pu]],
                                     osem[pb], add=True)
            if u >= 1:
                fire_scat()
            else:
                @pl.when(t > 0)
                def _():
                    fire_scat()
            @pl.when(j + NIB - 1 < NFULL)
            def _():
                off = pl.multiple_of((j + NIB - 1) * CH, 8)
                pltpu.async_copy(src_hbm.at[pl.ds(base + off, CH)],
                                 srci_v.at[pu], isem[pu])

        def group(t, _):
            for u in range(NIB):
                step(t, u)
            return 0
        lax.fori_loop(0, NFULL // NIB, group, 0)

        lastb = (NFULL - 1) % NRB
        lastu = (NFULL - 1) % NIB
        _wait_gath(lastb)
        _wait_didx(lastu)
        pltpu.async_copy(rows_v.at[lastb], acc_sh.at[dsti_v.at[lastu]],
                         ssem[lastb], add=True)
        if with_deg:
            pltpu.async_copy(ones_v.at[pl.ds(0, CH)],
                             deg_sh.at[dsti_v.at[lastu]], osem[lastb],
                             add=True)
        for b in range(NRB):
            _wait_scat(b)

        pltpu.async_copy(p_hbm.at[srcr_v.at[0]],
                         rows_v.at[0, pl.ds(0, REM)], gsem[0])
        pltpu.make_async_copy(p_hbm.at[srcr_v.at[0]],
                              rows_v.at[0, pl.ds(0, REM)], gsem[0]).wait()
        pltpu.sync_copy(rows_v.at[0, pl.ds(0, REM)],
                        acc_sh.at[dstr_v.at[0]], add=True)
        if with_deg:
            pltpu.sync_copy(ones_v.at[pl.ds(0, REM)],
                            deg_sh.at[dstr_v.at[0]], add=True)

        plsc.subcore_barrier()
        pltpu.sync_copy(acc_sh.at[pl.ds(row0, RPT)],
                        s_out.at[c, pl.ds(row0, RPT)])

        @pl.when(s == NS - 1)
        def _():
            pltpu.sync_copy(acc_sh.at[pl.ds(NS * RPT, RTAIL)],
                            s_out.at[c, pl.ds(NS * RPT, RTAIL)])

        if with_deg:
            pltpu.sync_copy(deg_sh.at[pl.ds(s * DZ, DZ)],
                            deg_out.at[c, 0, pl.ds(s * DZ, DZ)])

    return pl.kernel(body, mesh=mesh, out_type=out_type,
                     scratch_types=scratch, compiler_params=_SC_PARAMS)


@functools.cache
def _mk_sc_deg():
    mesh = plsc.VectorSubcoreMesh(core_axis_name="c", subcore_axis_name="s")
    out_type = jax.ShapeDtypeStruct((NW, 1, N_NODES), jnp.float32)
    scratch = [
        pltpu.VMEM((EPW,), jnp.int32),
        pltpu.VMEM((N_NODES,), jnp.float32),
    ]

    def body(dst_hbm, deg_out, dstf_v, deg_v):
        c = lax.axis_index("c")
        s = lax.axis_index("s")
        wid = s * NC + c
        pltpu.sync_copy(dst_hbm.at[pl.ds(wid * EPW, EPW)], dstf_v)

        zeros16 = jnp.zeros((LANES,), jnp.float32)

        def zdeg(i, _):
            deg_v[pl.ds(pl.multiple_of(i * LANES, 8), LANES)] = zeros16
            return 0
        lax.fori_loop(0, N_NODES // LANES, zdeg, 0)

        ones16 = jnp.ones((LANES,), jnp.float32)

        def dchunk(i, _):
            idx16 = dstf_v[pl.ds(pl.multiple_of(i * LANES, 8), LANES)]
            plsc.addupdate_scatter(deg_v, [idx16], ones16)
            return 0
        lax.fori_loop(0, EPW // LANES, dchunk, 0)
        pltpu.sync_copy(deg_v, deg_out.at[wid, 0])

    return pl.kernel(body, mesh=mesh, out_type=out_type,
                     scratch_types=scratch, compiler_params=_SC_PARAMS)


_DOT = functools.partial(jnp.dot, preferred_element_type=jnp.float32)


def _tc1_body(x_ref, wt_ref, wb_ref, b_ref, a_ref, p_ref):
    xb = x_ref[...]
    a_ref[...] = _DOT(xb, wt_ref[...]) + b_ref[...]
    p_ref[...] = _DOT(xb, wb_ref[...])


def _tc2_body(a1_ref, s_ref, deg_ref, wt_ref, wb_ref, b_ref, a2_ref, p2_ref):
    ssum = s_ref[0] + s_ref[1]
    deg = jnp.maximum(jnp.sum(deg_ref[...], axis=1, keepdims=True), 1.0)
    h = jnp.maximum(a1_ref[...] + ssum / deg, 0.0)
    a2_ref[...] = _DOT(h, wt_ref[...]) + b_ref[...]
    p2_ref[...] = _DOT(h, wb_ref[...])


def _tc3_body(a2_ref, s_ref, deg_ref, o_ref):
    ssum = s_ref[0] + s_ref[1]
    deg = jnp.maximum(jnp.sum(deg_ref[...], axis=1, keepdims=True), 1.0)
    o_ref[...] = jnp.maximum(a2_ref[...] + ssum / deg, 0.0)


_ROWS = pl.BlockSpec((BT, D), lambda i: (i, 0))
_WMAT = pl.BlockSpec((D, D), lambda i: (0, 0))
_BIAS = pl.BlockSpec((1, D), lambda i: (0, 0))
_SPART = pl.BlockSpec((NC, BT, D), lambda i: (0, i, 0))
_DEGP = pl.BlockSpec((BT, NW), lambda i: (i, 0))
_GRID = (N_NODES // BT,)
_ND = jax.ShapeDtypeStruct((N_NODES, D), jnp.float32)


def _tc1(x, wt, wb, b):
    return pl.pallas_call(
        _tc1_body, grid=_GRID,
        in_specs=[_ROWS, _WMAT, _WMAT, _BIAS],
        out_specs=[_ROWS, _ROWS], out_shape=[_ND, _ND],
    )(x, wt, wb, b)


def _tc2(a1, s_parts, deg_parts, wt, wb, b):
    return pl.pallas_call(
        _tc2_body, grid=_GRID,
        in_specs=[_ROWS, _SPART, _DEGP, _WMAT, _WMAT, _BIAS],
        out_specs=[_ROWS, _ROWS], out_shape=[_ND, _ND],
    )(a1, s_parts, deg_parts, wt, wb, b)


def _tc3(a2, s_parts, deg_parts):
    return pl.pallas_call(
        _tc3_body, grid=_GRID,
        in_specs=[_ROWS, _SPART, _DEGP],
        out_specs=_ROWS, out_shape=_ND,
    )(a2, s_parts, deg_parts)


def kernel(x, edge_index, W1, b1, W2, b2):
    src = edge_index[0].astype(jnp.int32)
    dst = edge_index[1].astype(jnp.int32)
    b1r = b1.reshape(1, D)
    b2r = b2.reshape(1, D)

    degp = _mk_sc_deg()(dst).reshape(NW, N_NODES).T
    a1, p1 = _tc1(x, W1[:D], W1[D:], b1r)
    (s1,) = _mk_sc_agg(False)(p1, src, dst)
    a2, p2 = _tc2(a1, s1, degp, W2[:D], W2[D:], b2r)
    (s2,) = _mk_sc_agg(False)(p2, src, dst)
    return _tc3(a2, s2, degp)


# speedup vs baseline: 1.1499x; 1.0499x over previous
import functools

import jax
import jax.numpy as jnp
from jax import lax
from jax.experimental import pallas as pl
from jax.experimental.pallas import tpu as pltpu
from jax.experimental.pallas import tpu_sc as plsc

N_NODES = 10000
N_EDGES = 320000
D = 128

NC = 2
NS = 16
NW = NC * NS
EPW = N_EDGES // NW
CH = 104
NFULL = 96
REM = EPW - NFULL * CH
RPT = 624
RTAIL = N_NODES - NS * RPT
BT = 1000
LANES = 16
NRB = 3
NIB = 6
assert NFULL % NIB == 0 and NIB % NRB == 0

_SC_PARAMS = pltpu.CompilerParams(needs_layout_passes=False)


DZ = 640
DN = NS * DZ


@functools.cache
def _mk_sc_agg(with_deg: bool):
    mesh = plsc.VectorSubcoreMesh(core_axis_name="c", subcore_axis_name="s")
    out_type = [jax.ShapeDtypeStruct((NC, N_NODES, D), jnp.float32)]
    scratch = [
        pltpu.VMEM((NRB, CH, D), jnp.float32),
        pltpu.VMEM((NIB, CH), jnp.int32),
        pltpu.VMEM((NIB, CH), jnp.int32),
        pltpu.VMEM((1, REM), jnp.int32),
        pltpu.VMEM((1, REM), jnp.int32),
        pltpu.VMEM_SHARED((N_NODES, D), jnp.float32),
    ]
    nsem = 2 * NRB + 2 * NIB
    if with_deg:
        out_type.append(jax.ShapeDtypeStruct((NC, 1, DN), jnp.float32))
        scratch += [
            pltpu.VMEM((112,), jnp.float32),
            pltpu.VMEM((DZ,), jnp.float32),
            pltpu.VMEM_SHARED((DN,), jnp.float32),
        ]
        nsem += NRB
    scratch += [pltpu.SemaphoreType.DMA] * nsem

    def body(p_hbm, src_hbm, dst_hbm, *rest):
        if with_deg:
            (s_out, deg_out, rows_v, srci_v, dsti_v, dstr_v, srcr_v, acc_sh,
             ones_v, zdeg_v, deg_sh, *sems) = rest
        else:
            (s_out, rows_v, srci_v, dsti_v, dstr_v, srcr_v, acc_sh,
             *sems) = rest
            deg_out = ones_v = zdeg_v = deg_sh = None
        gsem = sems[:NRB]
        ssem = sems[NRB:2 * NRB]
        isem = sems[2 * NRB:2 * NRB + NIB]
        dsem = sems[2 * NRB + NIB:2 * NRB + 2 * NIB]
        osem = sems[2 * NRB + 2 * NIB:]

        c = lax.axis_index("c")
        s = lax.axis_index("s")
        wid = s * NC + c
        base = wid * EPW

        pltpu.sync_copy(dst_hbm.at[pl.ds(base + NFULL * CH, REM)], dstr_v.at[0])
        pltpu.sync_copy(src_hbm.at[pl.ds(base + NFULL * CH, REM)], srcr_v.at[0])

        zeros16 = jnp.zeros((LANES,), jnp.float32)

        def zrow(i, _):
            for l in range(D // LANES):
                rows_v[0, i, pl.ds(l * LANES, LANES)] = zeros16
            return 0
        lax.fori_loop(0, CH, zrow, 0)

        assert RPT % CH == 0
        row0 = s * RPT
        for k in range(RPT // CH):
            pltpu.sync_copy(rows_v.at[0], acc_sh.at[pl.ds(row0 + k * CH, CH)])

        @pl.when(s == NS - 1)
        def _():
            pltpu.sync_copy(rows_v.at[0, pl.ds(0, RTAIL)],
                            acc_sh.at[pl.ds(NS * RPT, RTAIL)])

        if with_deg:
            ones16 = jnp.ones((LANES,), jnp.float32)
            for k in range(112 // LANES):
                ones_v[pl.ds(k * LANES, LANES)] = ones16
            for k in range(DZ // LANES):
                zdeg_v[pl.ds(k * LANES, LANES)] = zeros16

            pltpu.sync_copy(zdeg_v, deg_sh.at[pl.ds(s * DZ, DZ)])
        plsc.subcore_barrier()

        for u in range(NIB - 1):
            pltpu.async_copy(src_hbm.at[pl.ds(base + u * CH, CH)],
                             srci_v.at[u], isem[u])
        for u in range(NIB):
            pltpu.async_copy(dst_hbm.at[pl.ds(base + u * CH, CH)],
                             dsti_v.at[u], dsem[u])

        def _wait_scat(b):
            pltpu.make_async_copy(rows_v.at[b], acc_sh.at[dsti_v.at[0]],
                                  ssem[b]).wait()
            if with_deg:
                pltpu.make_async_copy(ones_v.at[pl.ds(0, CH)],
                                      deg_sh.at[dsti_v.at[0]],
                                      osem[b]).wait()

        def _wait_gath(b):
            pltpu.make_async_copy(p_hbm.at[srci_v.at[0]], rows_v.at[b],
                                  gsem[b]).wait()

        def _wait_idx(u):
            pltpu.make_async_copy(src_hbm.at[pl.ds(base, CH)], srci_v.at[u],
                                  isem[u]).wait()

        def _wait_didx(u):
            pltpu.make_async_copy(dst_hbm.at[pl.ds(base, CH)], dsti_v.at[u],
                                  dsem[u]).wait()

        def step(t, u):
            j = t * NIB + u
            rb = u % NRB
            fd = (u + NRB) % NIB

            def scat_done():
                _wait_scat(rb)

                @pl.when(j + NRB < NFULL)
                def _():
                    doff = pl.multiple_of((j + NRB) * CH, 8)
                    pltpu.async_copy(dst_hbm.at[pl.ds(base + doff, CH)],
                                     dsti_v.at[fd], dsem[fd])
            if u >= NRB:
                scat_done()
            else:
                @pl.when(t > 0)
                def _():
                    scat_done()
            _wait_idx(u)
            pltpu.async_copy(p_hbm.at[srci_v.at[u]], rows_v.at[rb], gsem[rb])
            pb = (u + NRB - 1) % NRB
            pu = (u + NIB - 1) % NIB

            def fire_scat():
                _wait_gath(pb)
                _wait_didx(pu)
                pltpu.async_copy(rows_v.at[pb], acc_sh.at[dsti_v.at[pu]],
                                 ssem[pb], add=True)
                if with_deg:
                    pltpu.async_copy(ones_v.at[pl.ds(0, CH)],
                                     deg_sh.at[dsti_v.at[pu]],
                                     osem[pb], add=True)
            if u >= 1:
                fire_scat()
            else:
                @pl.when(t > 0)
                def _():
                    fire_scat()
            @pl.when(j + NIB - 1 < NFULL)
            def _():
                off = pl.multiple_of((j + NIB - 1) * CH, 8)
                pltpu.async_copy(src_hbm.at[pl.ds(base + off, CH)],
                                 srci_v.at[pu], isem[pu])

        def group(t, _):
            for u in range(NIB):
                step(t, u)
            return 0
        lax.fori_loop(0, NFULL // NIB, group, 0)

        lastb = (NFULL - 1) % NRB
        lastu = (NFULL - 1) % NIB
        _wait_gath(lastb)
        _wait_didx(lastu)
        pltpu.async_copy(rows_v.at[lastb], acc_sh.at[dsti_v.at[lastu]],
                         ssem[lastb], add=True)
        if with_deg:
            pltpu.async_copy(ones_v.at[pl.ds(0, CH)],
                             deg_sh.at[dsti_v.at[lastu]], osem[lastb],
                             add=True)
        for b in range(NRB):
            _wait_scat(b)

        pltpu.async_copy(p_hbm.at[srcr_v.at[0]],
                         rows_v.at[0, pl.ds(0, REM)], gsem[0])
        pltpu.make_async_copy(p_hbm.at[srcr_v.at[0]],
                              rows_v.at[0, pl.ds(0, REM)], gsem[0]).wait()
        pltpu.sync_copy(rows_v.at[0, pl.ds(0, REM)],
                        acc_sh.at[dstr_v.at[0]], add=True)
        if with_deg:
            pltpu.sync_copy(ones_v.at[pl.ds(0, REM)],
                            deg_sh.at[dstr_v.at[0]], add=True)

        plsc.subcore_barrier()
        pltpu.sync_copy(acc_sh.at[pl.ds(row0, RPT)],
                        s_out.at[c, pl.ds(row0, RPT)])

        @pl.when(s == NS - 1)
        def _():
            pltpu.sync_copy(acc_sh.at[pl.ds(NS * RPT, RTAIL)],
                            s_out.at[c, pl.ds(NS * RPT, RTAIL)])

        if with_deg:
            pltpu.sync_copy(deg_sh.at[pl.ds(s * DZ, DZ)],
                            deg_out.at[c, 0, pl.ds(s * DZ, DZ)])

    return pl.kernel(body, mesh=mesh, out_type=out_type,
                     scratch_types=scratch, compiler_params=_SC_PARAMS)


@functools.cache
def _mk_sc_deg():
    mesh = plsc.VectorSubcoreMesh(core_axis_name="c", subcore_axis_name="s")
    out_type = jax.ShapeDtypeStruct((NW, 1, N_NODES), jnp.float32)
    scratch = [
        pltpu.VMEM((EPW,), jnp.int32),
        pltpu.VMEM((N_NODES,), jnp.float32),
    ]

    def body(dst_hbm, deg_out, dstf_v, deg_v):
        c = lax.axis_index("c")
        s = lax.axis_index("s")
        wid = s * NC + c
        pltpu.sync_copy(dst_hbm.at[pl.ds(wid * EPW, EPW)], dstf_v)

        zeros16 = jnp.zeros((LANES,), jnp.float32)

        def zdeg(i, _):
            deg_v[pl.ds(pl.multiple_of(i * LANES, 8), LANES)] = zeros16
            return 0
        lax.fori_loop(0, N_NODES // LANES, zdeg, 0)

        ones16 = jnp.ones((LANES,), jnp.float32)

        def dchunk(i, _):
            idx16 = dstf_v[pl.ds(pl.multiple_of(i * LANES, 8), LANES)]
            plsc.addupdate_scatter(deg_v, [idx16], ones16)
            return 0
        lax.fori_loop(0, EPW // LANES, dchunk, 0)
        pltpu.sync_copy(deg_v, deg_out.at[wid, 0])

    return pl.kernel(body, mesh=mesh, out_type=out_type,
                     scratch_types=scratch, compiler_params=_SC_PARAMS)


_DOT = functools.partial(jnp.dot, preferred_element_type=jnp.float32)


def _tc0_body(ei_ref, src_ref, dst_ref):
    src_ref[...] = ei_ref[0]
    dst_ref[...] = ei_ref[1]


def _tc0(ei):
    return pl.pallas_call(
        _tc0_body,
        out_shape=[jax.ShapeDtypeStruct((N_EDGES,), jnp.int32)] * 2,
    )(ei)


def _tc1_body(x_ref, wt_ref, wb_ref, b_ref, a_ref, p_ref):
    xb = x_ref[...]
    a_ref[...] = _DOT(xb, wt_ref[...]) + b_ref[...]
    p_ref[...] = _DOT(xb, wb_ref[...])


def _tc2_body(a1_ref, s_ref, deg_ref, wt_ref, wb_ref, b_ref, a2_ref, p2_ref):
    ssum = s_ref[0] + s_ref[1]
    deg = jnp.maximum(jnp.sum(deg_ref[...], axis=1, keepdims=True), 1.0)
    h = jnp.maximum(a1_ref[...] + ssum / deg, 0.0)
    a2_ref[...] = _DOT(h, wt_ref[...]) + b_ref[...]
    p2_ref[...] = _DOT(h, wb_ref[...])


def _tc3_body(a2_ref, s_ref, deg_ref, o_ref):
    ssum = s_ref[0] + s_ref[1]
    deg = jnp.maximum(jnp.sum(deg_ref[...], axis=1, keepdims=True), 1.0)
    o_ref[...] = jnp.maximum(a2_ref[...] + ssum / deg, 0.0)


_ROWS = pl.BlockSpec((BT, D), lambda i: (i, 0))
_WMAT = pl.BlockSpec((D, D), lambda i: (0, 0))
_BIAS = pl.BlockSpec((1, D), lambda i: (0, 0))
_SPART = pl.BlockSpec((NC, BT, D), lambda i: (0, i, 0))
_DEGP = pl.BlockSpec((BT, NW), lambda i: (i, 0))
_GRID = (N_NODES // BT,)
_ND = jax.ShapeDtypeStruct((N_NODES, D), jnp.float32)


def _tc1(x, wt, wb, b):
    return pl.pallas_call(
        _tc1_body, grid=_GRID,
        in_specs=[_ROWS, _WMAT, _WMAT, _BIAS],
        out_specs=[_ROWS, _ROWS], out_shape=[_ND, _ND],
    )(x, wt, wb, b)


def _tc2(a1, s_parts, deg_parts, wt, wb, b):
    return pl.pallas_call(
        _tc2_body, grid=_GRID,
        in_specs=[_ROWS, _SPART, _DEGP, _WMAT, _WMAT, _BIAS],
        out_specs=[_ROWS, _ROWS], out_shape=[_ND, _ND],
    )(a1, s_parts, deg_parts, wt, wb, b)


def _tc3(a2, s_parts, deg_parts):
    return pl.pallas_call(
        _tc3_body, grid=_GRID,
        in_specs=[_ROWS, _SPART, _DEGP],
        out_specs=_ROWS, out_shape=_ND,
    )(a2, s_parts, deg_parts)


def kernel(x, edge_index, W1, b1, W2, b2):
    ei = edge_index.astype(jnp.int32)
    b1r = b1.reshape(1, D)
    b2r = b2.reshape(1, D)

    src, dst = _tc0(ei)
    degp = _mk_sc_deg()(dst).reshape(NW, N_NODES).T
    a1, p1 = _tc1(x, W1[:D], W1[D:], b1r)
    (s1,) = _mk_sc_agg(False)(p1, src, dst)
    a2, p2 = _tc2(a1, s1, degp, W2[:D], W2[D:], b2r)
    (s2,) = _mk_sc_agg(False)(p2, src, dst)
    return _tc3(a2, s2, degp)


# speedup vs baseline: 1.1723x; 1.0195x over previous
import functools

import jax
import jax.numpy as jnp
from jax import lax
from jax.experimental import pallas as pl
from jax.experimental.pallas import tpu as pltpu
from jax.experimental.pallas import tpu_sc as plsc

N_NODES = 10000
N_EDGES = 320000
D = 128

NC = 2
NS = 16
NW = NC * NS
EPW = N_EDGES // NW
CH = 104
NFULL = 96
REM = EPW - NFULL * CH
RPT = 624
RTAIL = N_NODES - NS * RPT
BT = 2000
LANES = 16
NRB = 3
NIB = 6
assert NFULL % NIB == 0 and NIB % NRB == 0

_SC_PARAMS = pltpu.CompilerParams(needs_layout_passes=False)


DZ = 640
DN = NS * DZ


@functools.cache
def _mk_sc_agg(with_deg: bool):
    mesh = plsc.VectorSubcoreMesh(core_axis_name="c", subcore_axis_name="s")
    out_type = [jax.ShapeDtypeStruct((NC, N_NODES, D), jnp.float32)]
    scratch = [
        pltpu.VMEM((NRB, CH, D), jnp.float32),
        pltpu.VMEM((NIB, CH), jnp.int32),
        pltpu.VMEM((NIB, CH), jnp.int32),
        pltpu.VMEM((1, REM), jnp.int32),
        pltpu.VMEM((1, REM), jnp.int32),
        pltpu.VMEM_SHARED((N_NODES, D), jnp.float32),
    ]
    nsem = 2 * NRB + 2 * NIB
    if with_deg:
        out_type.append(jax.ShapeDtypeStruct((NC, 1, DN), jnp.float32))
        scratch += [
            pltpu.VMEM((112,), jnp.float32),
            pltpu.VMEM((DZ,), jnp.float32),
            pltpu.VMEM_SHARED((DN,), jnp.float32),
        ]
        nsem += NRB
    scratch += [pltpu.SemaphoreType.DMA] * nsem

    def body(p_hbm, src_hbm, dst_hbm, *rest):
        if with_deg:
            (s_out, deg_out, rows_v, srci_v, dsti_v, dstr_v, srcr_v, acc_sh,
             ones_v, zdeg_v, deg_sh, *sems) = rest
        else:
            (s_out, rows_v, srci_v, dsti_v, dstr_v, srcr_v, acc_sh,
             *sems) = rest
            deg_out = ones_v = zdeg_v = deg_sh = None
        gsem = sems[:NRB]
        ssem = sems[NRB:2 * NRB]
        isem = sems[2 * NRB:2 * NRB + NIB]
        dsem = sems[2 * NRB + NIB:2 * NRB + 2 * NIB]
        osem = sems[2 * NRB + 2 * NIB:]

        c = lax.axis_index("c")
        s = lax.axis_index("s")
        wid = s * NC + c
        base = wid * EPW

        pltpu.sync_copy(dst_hbm.at[pl.ds(base + NFULL * CH, REM)], dstr_v.at[0])
        pltpu.sync_copy(src_hbm.at[pl.ds(base + NFULL * CH, REM)], srcr_v.at[0])

        zeros16 = jnp.zeros((LANES,), jnp.float32)

        def zrow(i, _):
            for l in range(D // LANES):
                rows_v[0, i, pl.ds(l * LANES, LANES)] = zeros16
            return 0
        lax.fori_loop(0, CH, zrow, 0)

        row0 = s * RPT
        for k in range(RPT // CH):
            pltpu.sync_copy(rows_v.at[0], acc_sh.at[pl.ds(row0 + k * CH, CH)])
        if RPT % CH:
            pltpu.sync_copy(rows_v.at[0, pl.ds(0, RPT % CH)],
                            acc_sh.at[pl.ds(row0 + RPT - RPT % CH, RPT % CH)])

        @pl.when(s == NS - 1)
        def _():
            pltpu.sync_copy(rows_v.at[0, pl.ds(0, RTAIL)],
                            acc_sh.at[pl.ds(NS * RPT, RTAIL)])

        if with_deg:
            ones16 = jnp.ones((LANES,), jnp.float32)
            for k in range(112 // LANES):
                ones_v[pl.ds(k * LANES, LANES)] = ones16
            for k in range(DZ // LANES):
                zdeg_v[pl.ds(k * LANES, LANES)] = zeros16

            pltpu.sync_copy(zdeg_v, deg_sh.at[pl.ds(s * DZ, DZ)])
        plsc.subcore_barrier()

        for u in range(NIB - 1):
            pltpu.async_copy(src_hbm.at[pl.ds(base + u * CH, CH)],
                             srci_v.at[u], isem[u])
        for u in range(NIB):
            pltpu.async_copy(dst_hbm.at[pl.ds(base + u * CH, CH)],
                             dsti_v.at[u], dsem[u])

        def _wait_scat(b):
            pltpu.make_async_copy(rows_v.at[b], acc_sh.at[dsti_v.at[0]],
                                  ssem[b]).wait()
            if with_deg:
                pltpu.make_async_copy(ones_v.at[pl.ds(0, CH)],
                                      deg_sh.at[dsti_v.at[0]],
                                      osem[b]).wait()

        def _wait_gath(b):
            pltpu.make_async_copy(p_hbm.at[srci_v.at[0]], rows_v.at[b],
                                  gsem[b]).wait()

        def _wait_idx(u):
            pltpu.make_async_copy(src_hbm.at[pl.ds(base, CH)], srci_v.at[u],
                                  isem[u]).wait()

        def _wait_didx(u):
            pltpu.make_async_copy(dst_hbm.at[pl.ds(base, CH)], dsti_v.at[u],
                                  dsem[u]).wait()

        def step(t, u):
            j = t * NIB + u
            rb = u % NRB
            fd = (u + NRB) % NIB

            def scat_done():
                _wait_scat(rb)

                @pl.when(j + NRB < NFULL)
                def _():
                    doff = pl.multiple_of((j + NRB) * CH, 8)
                    pltpu.async_copy(dst_hbm.at[pl.ds(base + doff, CH)],
                                     dsti_v.at[fd], dsem[fd])
            if u >= NRB:
                scat_done()
            else:
                @pl.when(t > 0)
                def _():
                    scat_done()
            _wait_idx(u)
            pltpu.async_copy(p_hbm.at[srci_v.at[u]], rows_v.at[rb], gsem[rb])
            pb = (u + NRB - 1) % NRB
            pu = (u + NIB - 1) % NIB

            def fire_scat():
                _wait_gath(pb)
                _wait_didx(pu)
                pltpu.async_copy(rows_v.at[pb], acc_sh.at[dsti_v.at[pu]],
                                 ssem[pb], add=True)
                if with_deg:
                    pltpu.async_copy(ones_v.at[pl.ds(0, CH)],
                                     deg_sh.at[dsti_v.at[pu]],
                                     osem[pb], add=True)
            if u >= 1:
                fire_scat()
            else:
                @pl.when(t > 0)
                def _():
                    fire_scat()
            @pl.when(j + NIB - 1 < NFULL)
            def _():
                off = pl.multiple_of((j + NIB - 1) * CH, 8)
                pltpu.async_copy(src_hbm.at[pl.ds(base + off, CH)],
                                 srci_v.at[pu], isem[pu])

        def group(t, _):
            for u in range(NIB):
                step(t, u)
            return 0
        lax.fori_loop(0, NFULL // NIB, group, 0)

        lastb = (NFULL - 1) % NRB
        lastu = (NFULL - 1) % NIB
        _wait_gath(lastb)
        _wait_didx(lastu)
        pltpu.async_copy(rows_v.at[lastb], acc_sh.at[dsti_v.at[lastu]],
                         ssem[lastb], add=True)
        if with_deg:
            pltpu.async_copy(ones_v.at[pl.ds(0, CH)],
                             deg_sh.at[dsti_v.at[lastu]], osem[lastb],
                             add=True)
        for b in range(NRB):
            _wait_scat(b)

        pltpu.async_copy(p_hbm.at[srcr_v.at[0]],
                         rows_v.at[0, pl.ds(0, REM)], gsem[0])
        pltpu.make_async_copy(p_hbm.at[srcr_v.at[0]],
                              rows_v.at[0, pl.ds(0, REM)], gsem[0]).wait()
        pltpu.sync_copy(rows_v.at[0, pl.ds(0, REM)],
                        acc_sh.at[dstr_v.at[0]], add=True)
        if with_deg:
            pltpu.sync_copy(ones_v.at[pl.ds(0, REM)],
                            deg_sh.at[dstr_v.at[0]], add=True)

        plsc.subcore_barrier()
        pltpu.sync_copy(acc_sh.at[pl.ds(row0, RPT)],
                        s_out.at[c, pl.ds(row0, RPT)])

        @pl.when(s == NS - 1)
        def _():
            pltpu.sync_copy(acc_sh.at[pl.ds(NS * RPT, RTAIL)],
                            s_out.at[c, pl.ds(NS * RPT, RTAIL)])

        if with_deg:
            pltpu.sync_copy(deg_sh.at[pl.ds(s * DZ, DZ)],
                            deg_out.at[c, 0, pl.ds(s * DZ, DZ)])

    return pl.kernel(body, mesh=mesh, out_type=out_type,
                     scratch_types=scratch, compiler_params=_SC_PARAMS)


@functools.cache
def _mk_sc_deg():
    mesh = plsc.VectorSubcoreMesh(core_axis_name="c", subcore_axis_name="s")
    out_type = jax.ShapeDtypeStruct((NW, 1, N_NODES), jnp.float32)
    scratch = [
        pltpu.VMEM((EPW,), jnp.int32),
        pltpu.VMEM((N_NODES,), jnp.float32),
    ]

    def body(dst_hbm, deg_out, dstf_v, deg_v):
        c = lax.axis_index("c")
        s = lax.axis_index("s")
        wid = s * NC + c
        pltpu.sync_copy(dst_hbm.at[pl.ds(wid * EPW, EPW)], dstf_v)

        zeros16 = jnp.zeros((LANES,), jnp.float32)

        def zdeg(i, _):
            deg_v[pl.ds(pl.multiple_of(i * LANES, 8), LANES)] = zeros16
            return 0
        lax.fori_loop(0, N_NODES // LANES, zdeg, 0)

        ones16 = jnp.ones((LANES,), jnp.float32)

        def dchunk(i, _):
            idx16 = dstf_v[pl.ds(pl.multiple_of(i * LANES, 8), LANES)]
            plsc.addupdate_scatter(deg_v, [idx16], ones16)
            return 0
        lax.fori_loop(0, EPW // LANES, dchunk, 0)
        pltpu.sync_copy(deg_v, deg_out.at[wid, 0])

    return pl.kernel(body, mesh=mesh, out_type=out_type,
                     scratch_types=scratch, compiler_params=_SC_PARAMS)


_DOT = functools.partial(jnp.dot, preferred_element_type=jnp.float32)


def _tc0_body(ei_ref, src_ref, dst_ref):
    src_ref[...] = ei_ref[0]
    dst_ref[...] = ei_ref[1]


def _tc0(ei):
    return pl.pallas_call(
        _tc0_body,
        out_shape=[jax.ShapeDtypeStruct((N_EDGES,), jnp.int32)] * 2,
    )(ei)


def _tc1_body(x_ref, wt_ref, wb_ref, b_ref, a_ref, p_ref):
    xb = x_ref[...]
    a_ref[...] = _DOT(xb, wt_ref[...]) + b_ref[...]
    p_ref[...] = _DOT(xb, wb_ref[...])


def _tc2_body(a1_ref, s_ref, deg_ref, wt_ref, wb_ref, b_ref, a2_ref, p2_ref):
    ssum = s_ref[0] + s_ref[1]
    deg = jnp.maximum(jnp.sum(deg_ref[...], axis=1, keepdims=True), 1.0)
    h = jnp.maximum(a1_ref[...] + ssum / deg, 0.0)
    a2_ref[...] = _DOT(h, wt_ref[...]) + b_ref[...]
    p2_ref[...] = _DOT(h, wb_ref[...])


def _tc3_body(a2_ref, s_ref, deg_ref, o_ref):
    ssum = s_ref[0] + s_ref[1]
    deg = jnp.maximum(jnp.sum(deg_ref[...], axis=1, keepdims=True), 1.0)
    o_ref[...] = jnp.maximum(a2_ref[...] + ssum / deg, 0.0)


_ROWS = pl.BlockSpec((BT, D), lambda i: (i, 0))
_WMAT = pl.BlockSpec((D, D), lambda i: (0, 0))
_BIAS = pl.BlockSpec((1, D), lambda i: (0, 0))
_SPART = pl.BlockSpec((NC, BT, D), lambda i: (0, i, 0))
_DEGP = pl.BlockSpec((BT, NW), lambda i: (i, 0))
_GRID = (N_NODES // BT,)
_ND = jax.ShapeDtypeStruct((N_NODES, D), jnp.float32)


def _tc1(x, wt, wb, b):
    return pl.pallas_call(
        _tc1_body, grid=_GRID,
        in_specs=[_ROWS, _WMAT, _WMAT, _BIAS],
        out_specs=[_ROWS, _ROWS], out_shape=[_ND, _ND],
    )(x, wt, wb, b)


def _tc2(a1, s_parts, deg_parts, wt, wb, b):
    return pl.pallas_call(
        _tc2_body, grid=_GRID,
        in_specs=[_ROWS, _SPART, _DEGP, _WMAT, _WMAT, _BIAS],
        out_specs=[_ROWS, _ROWS], out_shape=[_ND, _ND],
    )(a1, s_parts, deg_parts, wt, wb, b)


def _tc3(a2, s_parts, deg_parts):
    return pl.pallas_call(
        _tc3_body, grid=_GRID,
        in_specs=[_ROWS, _SPART, _DEGP],
        out_specs=_ROWS, out_shape=_ND,
    )(a2, s_parts, deg_parts)


def kernel(x, edge_index, W1, b1, W2, b2):
    ei = edge_index.astype(jnp.int32)
    b1r = b1.reshape(1, D)
    b2r = b2.reshape(1, D)

    src, dst = _tc0(ei)
    degp = _mk_sc_deg()(dst).reshape(NW, N_NODES).T
    a1, p1 = _tc1(x, W1[:D], W1[D:], b1r)
    (s1,) = _mk_sc_agg(False)(p1, src, dst)
    a2, p2 = _tc2(a1, s1, degp, W2[:D], W2[D:], b2r)
    (s2,) = _mk_sc_agg(False)(p2, src, dst)
    return _tc3(a2, s2, degp)
